# baseline (speedup 1.0000x reference)
import functools

import jax
import jax.numpy as jnp
from jax import lax
from jax.experimental import pallas as pl
from jax.experimental.pallas import tpu as pltpu
from jax.experimental.pallas import tpu_sc as plsc

_BATCH = 4096
_HIST = 200
_D = 64
_NW = 32
_BB = _BATCH // _NW
_TS = 129

_mesh = plsc.VectorSubcoreMesh(core_axis_name="c", subcore_axis_name="s")


@functools.partial(
    pl.kernel,
    mesh=_mesh,
    out_type=jax.ShapeDtypeStruct((_HIST, _D // 8, _NW, 8, _BB), jnp.float32),
    scratch_types=[
        pltpu.VMEM((_HIST, _BB), jnp.int32),
        pltpu.VMEM((_BB, 2 * _D), jnp.float32),
        pltpu.VMEM((_BB, 2 * _D), jnp.float32),
        pltpu.VMEM((_D, _TS), jnp.float32),
        pltpu.VMEM((_D, _TS), jnp.float32),
        pltpu.SemaphoreType.DMA,
        pltpu.SemaphoreType.DMA,
        pltpu.SemaphoreType.DMA,
        pltpu.SemaphoreType.DMA,
    ],
    compiler_params=pltpu.CompilerParams(use_tc_tiling_on_sc=True,
                                         needs_layout_passes=False),
)
def _gather_kernel(xt_hbm, table_hbm, out_hbm, idx_all, rows0, rows1,
                   tr0, tr1, gs0, gs1, ss0, ss1):
    wid = lax.axis_index("s") * 2 + lax.axis_index("c")
    rows = (rows0, rows1)
    trs = (tr0, tr1)
    gs = (gs0, gs1)
    ss = (ss0, ss1)

    pltpu.sync_copy(xt_hbm.at[:, pl.ds(wid * _BB, _BB)], idx_all)

    iota16 = lax.iota(jnp.int32, 16)

    def g_copy(h, b):
        return pltpu.make_async_copy(
            table_hbm.at[idx_all.at[h]], rows[b], gs[b])

    def s_copies(h, b):
        return [
            pltpu.make_async_copy(
                trs[b].at[pl.ds(te * 8, 8), pl.ds(0, _BB)],
                out_hbm.at[h, te, wid], ss[b])
            for te in range(_D // 8)
        ]

    def transpose(b):
        rbuf = rows[b]
        tbuf = trs[b]

        @plsc.parallel_loop(0, _BB, unroll=4)
        def _bi(bi):
            cid = jnp.full((16,), 0, jnp.int32) + bi
            for j in range(_D // 16):
                vec = rbuf[bi, pl.ds(j * 16, 16)]
                plsc.store_scatter(tbuf, [iota16 + j * 16, cid], vec)

    g_copy(0, 0).start()

    @pl.loop(0, _HIST // 2)
    def _grp(gp):
        for b in range(2):
            h = gp * 2 + b

            @pl.when(h + 1 < _HIST)
            def _():
                g_copy(h + 1, 1 - b).start()

            g_copy(h, b).wait()

            @pl.when(h >= 2)
            def _():
                for c in s_copies(h - 2, b):
                    c.wait()

            transpose(b)
            for c in s_copies(h, b):
                c.start()

    for b, h in ((0, _HIST - 2), (1, _HIST - 1)):
        for c in s_copies(h, b):
            c.wait()


def kernel(x, table):
    xt = x.T.astype(jnp.int32)
    tpad = jnp.concatenate([table, table], axis=1)
    out5 = _gather_kernel(xt, tpad)
    return out5.transpose((2, 4, 0, 1, 3)).reshape(_BATCH, _HIST, _D)


# speedup vs baseline: 1.7671x; 1.7671x over previous
import functools

import jax
import jax.numpy as jnp
from jax import lax
from jax.experimental import pallas as pl
from jax.experimental.pallas import tpu as pltpu
from jax.experimental.pallas import tpu_sc as plsc

_BATCH = 4096
_HIST = 200
_D = 64
_NW = 32
_BB = _BATCH // _NW
_TS = 129

_mesh = plsc.VectorSubcoreMesh(core_axis_name="c", subcore_axis_name="s")


@functools.partial(
    pl.kernel,
    mesh=_mesh,
    out_type=jax.ShapeDtypeStruct((_HIST, _D // 8, _NW, 8, _BB), jnp.float32),
    scratch_types=[
        pltpu.VMEM((_HIST, _BB), jnp.int32),
        pltpu.VMEM((_BB, _D), jnp.float32),
        pltpu.VMEM((_BB, _D), jnp.float32),
        pltpu.VMEM((_D, _TS), jnp.float32),
        pltpu.VMEM((_D, _TS), jnp.float32),
        pltpu.SemaphoreType.DMA,
        pltpu.SemaphoreType.DMA,
        pltpu.SemaphoreType.DMA,
        pltpu.SemaphoreType.DMA,
    ],
    compiler_params=pltpu.CompilerParams(use_tc_tiling_on_sc=False,
                                         needs_layout_passes=False),
)
def _gather_kernel(xt_hbm, table_hbm, out_hbm, idx_all, rows0, rows1,
                   tr0, tr1, gs0, gs1, ss0, ss1):
    wid = lax.axis_index("s") * 2 + lax.axis_index("c")
    rows = (rows0, rows1)
    trs = (tr0, tr1)
    gs = (gs0, gs1)
    ss = (ss0, ss1)

    pltpu.sync_copy(xt_hbm.at[:, pl.ds(wid * _BB, _BB)], idx_all)

    iota16 = lax.iota(jnp.int32, 16)

    def g_copy(h, b):
        return pltpu.make_async_copy(
            table_hbm.at[idx_all.at[h]], rows[b], gs[b])

    def s_copies(h, b):
        return [
            pltpu.make_async_copy(
                trs[b].at[pl.ds(te * 8, 8), pl.ds(0, _BB)],
                out_hbm.at[h, te, wid], ss[b])
            for te in range(_D // 8)
        ]

    def transpose(b):
        rbuf = rows[b]
        tbuf = trs[b]

        @plsc.parallel_loop(0, _BB, unroll=4)
        def _bi(bi):
            cid = jnp.full((16,), 0, jnp.int32) + bi
            for j in range(_D // 16):
                vec = rbuf[bi, pl.ds(j * 16, 16)]
                plsc.store_scatter(tbuf, [iota16 + j * 16, cid], vec)

    g_copy(0, 0).start()

    @pl.loop(0, _HIST // 2)
    def _grp(gp):
        for b in range(2):
            h = gp * 2 + b

            @pl.when(h + 1 < _HIST)
            def _():
                g_copy(h + 1, 1 - b).start()

            g_copy(h, b).wait()

            @pl.when(h >= 2)
            def _():
                for c in s_copies(h - 2, b):
                    c.wait()

            transpose(b)
            for c in s_copies(h, b):
                c.start()

    for b, h in ((0, _HIST - 2), (1, _HIST - 1)):
        for c in s_copies(h, b):
            c.wait()


def kernel(x, table):
    xt = x.T.astype(jnp.int32)
    out5 = _gather_kernel(xt, table)
    return out5.transpose((2, 4, 0, 1, 3)).reshape(_BATCH, _HIST, _D)


# speedup vs baseline: 1.7710x; 1.0022x over previous
import functools

import jax
import jax.numpy as jnp
from jax import lax
from jax.experimental import pallas as pl
from jax.experimental.pallas import tpu as pltpu
from jax.experimental.pallas import tpu_sc as plsc

_BATCH = 4096
_HIST = 200
_D = 64
_NW = 32
_BB = _BATCH // _NW
_TS = 129

_mesh = plsc.VectorSubcoreMesh(core_axis_name="c", subcore_axis_name="s")


@functools.partial(
    pl.kernel,
    mesh=_mesh,
    out_type=jax.ShapeDtypeStruct((_HIST, _D // 8, _NW, 8, _BB), jnp.float32),
    scratch_types=[
        pltpu.VMEM((_HIST, _BB), jnp.int32),
        pltpu.VMEM((_BB, _D), jnp.float32),
        pltpu.VMEM((_BB, _D), jnp.float32),
        pltpu.VMEM((_D, _TS), jnp.float32),
        pltpu.VMEM((_D, _TS), jnp.float32),
        pltpu.SemaphoreType.DMA,
        pltpu.SemaphoreType.DMA,
        pltpu.SemaphoreType.DMA,
        pltpu.SemaphoreType.DMA,
    ],
    compiler_params=pltpu.CompilerParams(use_tc_tiling_on_sc=False,
                                         needs_layout_passes=False),
)
def _gather_kernel(xt_hbm, table_hbm, out_hbm, idx_all, rows0, rows1,
                   tr0, tr1, gs0, gs1, ss0, ss1):
    wid = lax.axis_index("s") * 2 + lax.axis_index("c")
    rows = (rows0, rows1)
    trs = (tr0, tr1)
    gs = (gs0, gs1)
    ss = (ss0, ss1)

    pltpu.sync_copy(xt_hbm.at[:, pl.ds(wid * _BB, _BB)], idx_all)

    iota16 = lax.iota(jnp.int32, 16)

    def g_copy(h, b):
        return pltpu.make_async_copy(
            table_hbm.at[idx_all.at[h]], rows[b], gs[b])

    def s_copies(h, b):
        return [
            pltpu.make_async_copy(
                trs[b].at[pl.ds(te * 8, 8), pl.ds(0, _BB)],
                out_hbm.at[h, te, wid], ss[b])
            for te in range(_D // 8)
        ]

    def transpose(b):
        rbuf = rows[b]
        tbuf = trs[b]

        @plsc.parallel_loop(0, _BB, unroll=8)
        def _bi(bi):
            cid = jnp.full((16,), 0, jnp.int32) + bi
            for j in range(_D // 16):
                vec = rbuf[bi, pl.ds(j * 16, 16)]
                plsc.store_scatter(tbuf, [iota16 + j * 16, cid], vec)

    g_copy(0, 0).start()

    @pl.loop(0, _HIST // 2)
    def _grp(gp):
        for b in range(2):
            h = gp * 2 + b

            @pl.when(h + 1 < _HIST)
            def _():
                g_copy(h + 1, 1 - b).start()

            g_copy(h, b).wait()

            @pl.when(h >= 2)
            def _():
                for c in s_copies(h - 2, b):
                    c.wait()

            transpose(b)
            for c in s_copies(h, b):
                c.start()

    for b, h in ((0, _HIST - 2), (1, _HIST - 1)):
        for c in s_copies(h, b):
            c.wait()


def kernel(x, table):
    xt = x.T.astype(jnp.int32)
    out5 = _gather_kernel(xt, table)
    return out5.transpose((2, 4, 0, 1, 3)).reshape(_BATCH, _HIST, _D)
